# baseline (speedup 1.0000x reference)
import functools

import jax
import jax.numpy as jnp
from jax import lax
from jax.experimental import pallas as pl
from jax.experimental.pallas import tpu as pltpu
from jax.experimental.pallas import tpu_sc as plsc

N = 10000
E = 320000
D = 128
NC = 2
NS = 16
NW = NC * NS
EPT = E // NW
C = 80
KC = EPT // C
SCH = 25
NSC = KC // SCH
RB = 80
NB = N // RB
BPS = -(-NB // NS)
DG = D // 16

_mesh = plsc.VectorSubcoreMesh(core_axis_name="c", subcore_axis_name="s")
_f32 = jnp.float32


def _worker_id():
    c = lax.axis_index("c")
    s = lax.axis_index("s")
    return c, s, s * NC + c


def _for_blocks(s, fn):
    for i in range(BPS):
        blk = s + i * NS
        if (i + 1) * NS <= NB:
            fn(blk)
        else:
            @pl.when(blk < NB)
            def _():
                fn(blk)


def _zero_acc(zbuf, acc, s, zrows):
    def zrow(r, _):
        for g in range(DG):
            zbuf[r, pl.ds(g * 16, 16)] = jnp.zeros((16,), _f32)
        return 0
    lax.fori_loop(0, zrows, zrow, 0)
    zsrc = zbuf.at[pl.ds(0, zrows)]

    def one(blk):
        for k in range(RB // zrows):
            pltpu.sync_copy(zsrc, acc.at[pl.ds(blk * RB + k * zrows, zrows)])
    _for_blocks(s, one)


def _edge_loop(src_hbm, row4, col4, w4, acc, rowv, colv, wv, bufs, gsems,
               ssems, wid):
    def g_start(i, b):
        pltpu.async_copy(src_hbm.at[colv.at[i]], bufs[b], gsems[b])

    def g_wait(i, b):
        pltpu.make_async_copy(src_hbm.at[colv.at[i]], bufs[b], gsems[b]).wait()

    def s_start(i, b):
        pltpu.async_copy(bufs[b], acc.at[rowv.at[i]], ssems[b], add=True)

    def s_wait(i, b):
        pltpu.make_async_copy(bufs[b], acc.at[rowv.at[i]], ssems[b]).wait()

    def compute(i, b):
        buf = bufs[b]

        def edge(e, _):
            w = wv[i, pl.ds(e, 16)][0]
            for g in range(DG):
                sl = pl.ds(g * 16, 16)
                buf[e, sl] = buf[e, sl] * w
            return 0

        @plsc.parallel_loop(0, C, unroll=4)
        def _(e):
            edge(e, 0)

    def step(i, b, wait_prev, nxt):
        g_wait(i, b)
        compute(i, b)
        s_start(i, b)
        if wait_prev is not None:
            s_wait(wait_prev, (b + 2) % 3)
        if nxt is not None:
            g_start(nxt, (b + 2) % 3)

    def superchunk(sc, _):
        pltpu.sync_copy(row4.at[wid, sc], rowv)
        pltpu.sync_copy(col4.at[wid, sc], colv)
        pltpu.sync_copy(w4.at[wid, sc], wv.at[pl.ds(0, SCH)])

        g_start(0, 0)
        g_start(1, 1)
        step(0, 0, None, 2)
        step(1, 1, 0, 3)
        step(2, 2, 1, 4)

        def tri(g, _):
            i = 3 * g
            step(i, 0, i - 1, i + 2)
            step(i + 1, 1, i, i + 3)
            step(i + 2, 2, i + 1, i + 4)
            return 0

        lax.fori_loop(1, (SCH - 4) // 3, tri, 0)
        step(SCH - 4, 0, SCH - 5, SCH - 2)
        step(SCH - 3, 1, SCH - 4, SCH - 1)
        step(SCH - 2, 2, SCH - 3, None)
        step(SCH - 1, 0, None, None)
        s_wait(SCH - 2, 2)
        s_wait(SCH - 1, 0)
        return 0

    lax.fori_loop(0, NSC, superchunk, 0)


def _writeout(acc, out, c, s):
    def one(blk):
        rows = pl.ds(blk * RB, RB)
        pltpu.sync_copy(acc.at[rows], out.at[c, rows])
    _for_blocks(s, one)


def _spmm(table_hbm, row4, col4, w4, p_out, acc, rowv, colv, wv, b0, b1, b2,
          g0, g1, g2, s0, s1, s2):
    c, s, wid = _worker_id()
    _zero_acc(b0, acc, s, RB)
    plsc.subcore_barrier()
    _edge_loop(table_hbm, row4, col4, w4, acc, rowv, colv, wv, (b0, b1, b2),
               (g0, g1, g2), (s0, s1, s2), wid)
    plsc.subcore_barrier()
    _writeout(acc, p_out, c, s)


def _spmm_sc(table_hbm, row4, col4, w4):
    return pl.kernel(
        _spmm,
        out_type=jax.ShapeDtypeStruct((NC, N, D), _f32),
        mesh=_mesh,
        scratch_types=[
            pltpu.VMEM_SHARED((N, D), _f32),
            pltpu.VMEM((SCH, C), jnp.int32),
            pltpu.VMEM((SCH, C), jnp.int32),
            pltpu.VMEM((SCH + 1, C), _f32),
            pltpu.VMEM((C, D), _f32),
            pltpu.VMEM((C, D), _f32),
            pltpu.VMEM((C, D), _f32),
            pltpu.SemaphoreType.DMA,
            pltpu.SemaphoreType.DMA,
            pltpu.SemaphoreType.DMA,
            pltpu.SemaphoreType.DMA,
            pltpu.SemaphoreType.DMA,
            pltpu.SemaphoreType.DMA,
        ],
    )(table_hbm, row4, col4, w4)


TB = 400


def _combine_tc(p):
    def body(a_ref, b_ref, o_ref):
        o_ref[...] = a_ref[0] + b_ref[0]

    return pl.pallas_call(
        body,
        grid=(N // TB,),
        in_specs=[
            pl.BlockSpec((1, TB, D), lambda i: (0, i, 0)),
            pl.BlockSpec((1, TB, D), lambda i: (1, i, 0)),
        ],
        out_specs=pl.BlockSpec((TB, D), lambda i: (i, 0)),
        out_shape=jax.ShapeDtypeStruct((N, D), _f32),
    )(p, p)


def _final_tc(q, x, alpha):
    def body(a_ref, b_ref, x_ref, al_ref, o_ref):
        sg = jax.nn.sigmoid(al_ref[...])
        o_ref[...] = (a_ref[0] + b_ref[0]) * sg - x_ref[...]

    return pl.pallas_call(
        body,
        grid=(N // TB,),
        in_specs=[
            pl.BlockSpec((1, TB, D), lambda i: (0, i, 0)),
            pl.BlockSpec((1, TB, D), lambda i: (1, i, 0)),
            pl.BlockSpec((TB, D), lambda i: (i, 0)),
            pl.BlockSpec((TB, 1), lambda i: (i, 0)),
        ],
        out_specs=pl.BlockSpec((TB, D), lambda i: (i, 0)),
        out_shape=jax.ShapeDtypeStruct((N, D), _f32),
    )(q, q, x, alpha.reshape(N, 1))


def kernel(t, x, edge_index, edge_weight, alpha):
    row4 = edge_index[0].reshape(NW, NSC, SCH, C)
    col4 = edge_index[1].reshape(NW, NSC, SCH, C)
    w4 = edge_weight.reshape(NW, NSC, SCH, C)
    p = _spmm_sc(x, row4, col4, w4)
    ax = _combine_tc(p)
    q = _spmm_sc(ax, row4, col4, w4)
    return _final_tc(q, x, alpha)


# speedup vs baseline: 1.0372x; 1.0372x over previous
import functools

import jax
import jax.numpy as jnp
from jax import lax
from jax.experimental import pallas as pl
from jax.experimental.pallas import tpu as pltpu
from jax.experimental.pallas import tpu_sc as plsc

N = 10000
E = 320000
D = 128
NC = 2
NS = 16
NW = NC * NS
EPT = E // NW
C = 80
KC = EPT // C
SCH = 25
NSC = KC // SCH
RB = 80
NB = N // RB
BPS = -(-NB // NS)
DG = D // 16

_mesh = plsc.VectorSubcoreMesh(core_axis_name="c", subcore_axis_name="s")
_f32 = jnp.float32


def _worker_id():
    c = lax.axis_index("c")
    s = lax.axis_index("s")
    return c, s, s * NC + c


def _for_blocks(s, fn):
    for i in range(BPS):
        blk = s + i * NS
        if (i + 1) * NS <= NB:
            fn(blk)
        else:
            @pl.when(blk < NB)
            def _():
                fn(blk)


def _zero_acc(zbuf, acc, s, zrows):
    def zrow(r, _):
        for g in range(DG):
            zbuf[r, pl.ds(g * 16, 16)] = jnp.zeros((16,), _f32)
        return 0
    lax.fori_loop(0, zrows, zrow, 0)
    zsrc = zbuf.at[pl.ds(0, zrows)]

    def one(blk):
        for k in range(RB // zrows):
            pltpu.sync_copy(zsrc, acc.at[pl.ds(blk * RB + k * zrows, zrows)])
    _for_blocks(s, one)


def _edge_loop(src_hbm, row4, col4, w4, acc, rowv, colv, wv, bufs, gsems,
               ssems, wid):
    def g_start(i, b):
        pltpu.async_copy(src_hbm.at[colv.at[i]], bufs[b], gsems[b])

    def g_wait(i, b):
        pltpu.make_async_copy(src_hbm.at[colv.at[i]], bufs[b], gsems[b]).wait()

    def s_start(i, b):
        pltpu.async_copy(bufs[b], acc.at[rowv.at[i]], ssems[b], add=True)

    def s_wait(i, b):
        pltpu.make_async_copy(bufs[b], acc.at[rowv.at[i]], ssems[b]).wait()

    def compute(i, b):
        buf = bufs[b]

        def edge(e, _):
            w = wv[i, pl.ds(e, 16)][0]
            for g in range(DG):
                sl = pl.ds(g * 16, 16)
                buf[e, sl] = buf[e, sl] * w
            return 0

        @plsc.parallel_loop(0, C, unroll=2)
        def _(e):
            edge(e, 0)

    def step(i, b, wait_prev, nxt):
        g_wait(i, b)
        compute(i, b)
        s_start(i, b)
        if wait_prev is not None:
            s_wait(wait_prev, (b + 2) % 3)
        if nxt is not None:
            g_start(nxt, (b + 2) % 3)

    def superchunk(sc, _):
        stsem = gsems[0]
        pltpu.async_copy(row4.at[wid, sc], rowv, stsem)
        pltpu.async_copy(col4.at[wid, sc], colv, stsem)
        pltpu.async_copy(w4.at[wid, sc], wv.at[pl.ds(0, SCH)], stsem)
        pltpu.make_async_copy(row4.at[wid, sc], rowv, stsem).wait()
        pltpu.make_async_copy(col4.at[wid, sc], colv, stsem).wait()
        pltpu.make_async_copy(w4.at[wid, sc], wv.at[pl.ds(0, SCH)], stsem).wait()

        g_start(0, 0)
        g_start(1, 1)
        step(0, 0, None, 2)
        step(1, 1, 0, 3)
        step(2, 2, 1, 4)

        def tri(g, _):
            i = 3 * g
            step(i, 0, i - 1, i + 2)
            step(i + 1, 1, i, i + 3)
            step(i + 2, 2, i + 1, i + 4)
            return 0

        lax.fori_loop(1, (SCH - 4) // 3, tri, 0)
        step(SCH - 4, 0, SCH - 5, SCH - 2)
        step(SCH - 3, 1, SCH - 4, SCH - 1)
        step(SCH - 2, 2, SCH - 3, None)
        step(SCH - 1, 0, None, None)
        s_wait(SCH - 2, 2)
        s_wait(SCH - 1, 0)
        return 0

    lax.fori_loop(0, NSC, superchunk, 0)


def _writeout(acc, out, c, s):
    def one(blk):
        rows = pl.ds(blk * RB, RB)
        pltpu.sync_copy(acc.at[rows], out.at[c, rows])
    _for_blocks(s, one)


def _spmm(table_hbm, row4, col4, w4, p_out, acc, rowv, colv, wv, b0, b1, b2,
          g0, g1, g2, s0, s1, s2):
    c, s, wid = _worker_id()
    _zero_acc(b0, acc, s, RB)
    plsc.subcore_barrier()
    _edge_loop(table_hbm, row4, col4, w4, acc, rowv, colv, wv, (b0, b1, b2),
               (g0, g1, g2), (s0, s1, s2), wid)
    plsc.subcore_barrier()
    _writeout(acc, p_out, c, s)


def _spmm_sc(table_hbm, row4, col4, w4):
    return pl.kernel(
        _spmm,
        out_type=jax.ShapeDtypeStruct((NC, N, D), _f32),
        mesh=_mesh,
        scratch_types=[
            pltpu.VMEM_SHARED((N, D), _f32),
            pltpu.VMEM((SCH, C), jnp.int32),
            pltpu.VMEM((SCH, C), jnp.int32),
            pltpu.VMEM((SCH + 1, C), _f32),
            pltpu.VMEM((C, D), _f32),
            pltpu.VMEM((C, D), _f32),
            pltpu.VMEM((C, D), _f32),
            pltpu.SemaphoreType.DMA,
            pltpu.SemaphoreType.DMA,
            pltpu.SemaphoreType.DMA,
            pltpu.SemaphoreType.DMA,
            pltpu.SemaphoreType.DMA,
            pltpu.SemaphoreType.DMA,
        ],
    )(table_hbm, row4, col4, w4)


TB = 400


def _combine_tc(p):
    def body(a_ref, b_ref, o_ref):
        o_ref[...] = a_ref[0] + b_ref[0]

    return pl.pallas_call(
        body,
        grid=(N // TB,),
        in_specs=[
            pl.BlockSpec((1, TB, D), lambda i: (0, i, 0)),
            pl.BlockSpec((1, TB, D), lambda i: (1, i, 0)),
        ],
        out_specs=pl.BlockSpec((TB, D), lambda i: (i, 0)),
        out_shape=jax.ShapeDtypeStruct((N, D), _f32),
    )(p, p)


def _final_tc(q, x, alpha):
    def body(a_ref, b_ref, x_ref, al_ref, o_ref):
        sg = jax.nn.sigmoid(al_ref[...])
        o_ref[...] = (a_ref[0] + b_ref[0]) * sg - x_ref[...]

    return pl.pallas_call(
        body,
        grid=(N // TB,),
        in_specs=[
            pl.BlockSpec((1, TB, D), lambda i: (0, i, 0)),
            pl.BlockSpec((1, TB, D), lambda i: (1, i, 0)),
            pl.BlockSpec((TB, D), lambda i: (i, 0)),
            pl.BlockSpec((TB, 1), lambda i: (i, 0)),
        ],
        out_specs=pl.BlockSpec((TB, D), lambda i: (i, 0)),
        out_shape=jax.ShapeDtypeStruct((N, D), _f32),
    )(q, q, x, alpha.reshape(N, 1))


def kernel(t, x, edge_index, edge_weight, alpha):
    row4 = edge_index[0].reshape(NW, NSC, SCH, C)
    col4 = edge_index[1].reshape(NW, NSC, SCH, C)
    w4 = edge_weight.reshape(NW, NSC, SCH, C)
    p = _spmm_sc(x, row4, col4, w4)
    ax = _combine_tc(p)
    q = _spmm_sc(ax, row4, col4, w4)
    return _final_tc(q, x, alpha)
